# baseline (speedup 1.0000x reference)
import functools

import jax
import jax.numpy as jnp
from jax import lax
from jax.experimental import pallas as pl
from jax.experimental.pallas import tpu as pltpu
from jax.experimental.pallas import tpu_sc as plsc

NC = 2
NS = 16
NW = NC * NS
LANES = 16


def _repack_bias_body(V, BLK, sym_t_ref, pos_t_ref, table_ref, bias_ref):
    i = pl.program_id(0)
    blk = sym_t_ref[...]
    table_ref[...] = jnp.transpose(blk)

    @pl.when(i == 0)
    def _():
        bias_ref[...] = jnp.transpose(pos_t_ref[...])

    col = jax.lax.broadcasted_iota(jnp.int32, blk.shape, 1)
    valid = col < (V - i * BLK)
    part = jnp.sum(jnp.where(valid, blk, 0.0), axis=1)
    bias_ref[...] = bias_ref[...] - part[None, :] * (1.0 / V)


@functools.partial(jax.jit, static_argnames=("B", "S", "D"))
def _sc_embed(idx_flat, sym_lin, bias, *, B, S, D):
    SPAN = B * S // NW
    CH = 128
    NCHUNK = SPAN // CH
    NBUF = 4
    nvec = D // LANES
    mesh = plsc.VectorSubcoreMesh(
        core_axis_name="c", subcore_axis_name="s", num_cores=NC, num_subcores=NS
    )

    scratch = [pltpu.VMEM((2 * S, D), jnp.float32)]
    scratch += [pltpu.VMEM((CH,), jnp.int32) for _ in range(NBUF)]
    scratch += [pltpu.VMEM((CH, D), jnp.float32) for _ in range(NBUF)]
    scratch += [pltpu.SemaphoreType.DMA for _ in range(3 * NBUF)]

    @functools.partial(
        pl.kernel,
        out_type=jax.ShapeDtypeStruct((B * S, D), jnp.float32),
        mesh=mesh,
        scratch_types=scratch,
        compiler_params=pltpu.CompilerParams(use_tc_tiling_on_sc=False),
    )
    def body(idx_hbm, sym_hbm, bias_hbm, out_hbm, bias_v, *bufs):
        idxb = bufs[0:NBUF]
        rowsb = bufs[NBUF:2 * NBUF]
        gsem = bufs[2 * NBUF:3 * NBUF]
        wsem = bufs[3 * NBUF:4 * NBUF]
        isem = bufs[4 * NBUF:5 * NBUF]
        wid = lax.axis_index("s") * NC + lax.axis_index("c")
        base0 = wid * SPAN
        pltpu.sync_copy(bias_hbm, bias_v.at[pl.ds(0, S)])
        pltpu.sync_copy(bias_hbm, bias_v.at[pl.ds(S, S)])

        gcp = [None] * NCHUNK
        wcp = [None] * NCHUNK
        icp = [None] * NCHUNK

        def load_idx(k):
            b = k % NBUF
            off = base0 + k * CH
            icp[k] = pltpu.async_copy(idx_hbm.at[pl.ds(off, CH)], idxb[b], isem[b])

        def start(k):
            b = k % NBUF
            if k >= NBUF:
                wcp[k - NBUF].wait()
            icp[k].wait()
            gcp[k] = pltpu.async_copy(sym_hbm.at[idxb[b]], rowsb[b], gsem[b])
            if k + 1 < NCHUNK:
                load_idx(k + 1)

        def finish(k):
            b = k % NBUF
            gcp[k].wait()
            s0 = (k * CH) % S

            def add(r, c2):
                for c in range(nvec):
                    sl = pl.ds(c * LANES, LANES)
                    rowsb[b][r, sl] = rowsb[b][r, sl] + bias_v[s0 + r, sl]
                return c2

            lax.fori_loop(0, CH, add, 0)
            off = base0 + k * CH
            wcp[k] = pltpu.async_copy(rowsb[b], out_hbm.at[pl.ds(off, CH)], wsem[b])

        load_idx(0)
        start(0)
        for k in range(1, NCHUNK):
            start(k)
            finish(k - 1)
        finish(NCHUNK - 1)
        for k in range(NCHUNK - NBUF, NCHUNK):
            wcp[k].wait()

    return body(idx_flat, sym_lin, bias)


def kernel(inputs, sym_table, pos_table):
    B, S = inputs.shape
    V, D = sym_table.shape
    sym_t = sym_table.T
    pos_t = pos_table[:S].T

    BLK = 4096
    sym_lin, bias = pl.pallas_call(
        functools.partial(_repack_bias_body, V, BLK),
        out_shape=[
            jax.ShapeDtypeStruct((V, D), jnp.float32),
            jax.ShapeDtypeStruct((S, D), jnp.float32),
        ],
        grid=(pl.cdiv(V, BLK),),
        in_specs=[
            pl.BlockSpec((D, BLK), lambda i: (0, i)),
            pl.BlockSpec((D, S), lambda i: (0, 0)),
        ],
        out_specs=[
            pl.BlockSpec((BLK, D), lambda i: (i, 0)),
            pl.BlockSpec((S, D), lambda i: (0, 0)),
        ],
    )(sym_t, pos_t)

    idx_flat = inputs.reshape(-1).astype(jnp.int32)
    out = _sc_embed(idx_flat, sym_lin, bias, B=B, S=S, D=D)
    return out.reshape(B, S, D)
